# baseline (speedup 1.0000x reference)
import functools

import jax
import jax.numpy as jnp
from jax import lax
from jax.experimental import pallas as pl
from jax.experimental.pallas import tpu as pltpu
from jax.experimental.pallas import tpu_sc as plsc
from jax._src.pallas import mpmd as pl_mpmd

C = 100000
S = 4
D = 64
B = 16384
G = C * S
MOMENTUM = 0.99

NC = 2
NS = 16
NW = NC * NS
CHUNK = 128

B_PER_W = B // NW
ROWS_PER_W = G // NW
CNT_PER_TILE = G // NS
B_PER_TILE = B // NS

_MESH = dict(core_axis_name="c", subcore_axis_name="s")


def _wid():
    return lax.axis_index("s") * NC + lax.axis_index("c")


def _copy_body(protos_hbm, counts_hbm, idx2d_hbm, ones_hbm,
               protos_out, counts_out,
               idx_v, ones_v, cnt_stage, spmem):
    wid = _wid()
    r0 = wid * ROWS_PER_W
    pltpu.sync_copy(protos_hbm.at[pl.ds(r0, ROWS_PER_W), :],
                    protos_out.at[pl.ds(r0, ROWS_PER_W), :])

    cid = lax.axis_index("c")
    sid = lax.axis_index("s")

    @pl.when(cid == 0)
    def _stage_in():
        c0 = sid * CNT_PER_TILE
        pltpu.sync_copy(counts_hbm.at[pl.ds(c0, CNT_PER_TILE)],
                        spmem.at[pl.ds(c0, CNT_PER_TILE)])

    plsc.subcore_barrier()

    @pl.when(cid == 0)
    def _scatter_add():
        pltpu.sync_copy(ones_hbm, ones_v)
        pltpu.sync_copy(idx2d_hbm.at[pl.ds(sid * (B_PER_TILE // CHUNK),
                                           B_PER_TILE // CHUNK), :], idx_v)
        for j in range(B_PER_TILE // CHUNK):
            pltpu.sync_copy(ones_v, spmem.at[idx_v.at[j]], add=True)

    plsc.subcore_barrier()

    @pl.when(cid == 0)
    def _stage_out():
        c0 = sid * CNT_PER_TILE
        pltpu.sync_copy(spmem.at[pl.ds(c0, CNT_PER_TILE)],
                        counts_out.at[pl.ds(c0, CNT_PER_TILE)])


_k_copy = pl.kernel(
    _copy_body,
    out_type=(
        jax.ShapeDtypeStruct((G, D), jnp.float32),
        jax.ShapeDtypeStruct((G,), jnp.float32),
    ),
    mesh=plsc.VectorSubcoreMesh(**_MESH),
    compiler_params=pltpu.CompilerParams(use_tc_tiling_on_sc=False),
    scratch_types=[
        pltpu.VMEM((B_PER_TILE // CHUNK, CHUNK), jnp.int32),
        pltpu.VMEM((CHUNK,), jnp.float32),
        pltpu.VMEM((CHUNK,), jnp.float32),
        pltpu.VMEM_SHARED((G,), jnp.float32),
    ],
)


def _gather_body(protos_hbm, idx2d_hbm, rows_out, idx_v, rows_v, sem):
    wid = _wid()
    n_chunks = B_PER_W // CHUNK
    pltpu.sync_copy(idx2d_hbm.at[pl.ds(wid * n_chunks, n_chunks), :], idx_v)
    descs = []
    for j in range(n_chunks):
        descs.append(pltpu.async_copy(
            protos_hbm.at[idx_v.at[j]],
            rows_v.at[pl.ds(j * CHUNK, CHUNK), :], sem))
    for d in descs:
        d.wait()
    pltpu.sync_copy(rows_v, rows_out.at[pl.ds(wid * B_PER_W, B_PER_W), :])


_k_gather = pl.kernel(
    _gather_body,
    out_type=jax.ShapeDtypeStruct((B, D), jnp.float32),
    mesh=plsc.VectorSubcoreMesh(**_MESH),
    compiler_params=pltpu.CompilerParams(use_tc_tiling_on_sc=False),
    scratch_types=[
        pltpu.VMEM((B_PER_W // CHUNK, CHUNK), jnp.int32),
        pltpu.VMEM((B_PER_W, D), jnp.float32),
        pltpu.SemaphoreType.DMA,
    ],
)


BLK_I = 1024
BLK_J = 1024
NI = B // BLK_I
NJ = B // BLK_J


def _mm_body(ids_col, ids_row, feats, prows, out, cnt):
    j = pl.program_id(1)

    @pl.when(j == 0)
    def _init():
        out[...] = jnp.zeros_like(out)
        cnt[...] = jnp.zeros_like(cnt)

    mask = (ids_col[...] == ids_row[...]).astype(jnp.float32)
    out[...] += jnp.dot(mask, feats[...], preferred_element_type=jnp.float32)
    cnt[...] += jnp.sum(mask, axis=1, keepdims=True)

    @pl.when(j == NJ - 1)
    def _finalize():
        mean = out[...] / cnt[...]
        out[...] = MOMENTUM * prows[...] + (1.0 - MOMENTUM) * mean


_k_mm = pl.pallas_call(
    _mm_body,
    grid=(NI, NJ),
    in_specs=[
        pl.BlockSpec((BLK_I, 1), lambda i, j: (i, 0)),
        pl.BlockSpec((1, BLK_J), lambda i, j: (0, j)),
        pl.BlockSpec((BLK_J, D), lambda i, j: (j, 0)),
        pl.BlockSpec((BLK_I, D), lambda i, j: (i, 0)),
    ],
    out_specs=pl.BlockSpec((BLK_I, D), lambda i, j: (i, 0)),
    out_shape=jax.ShapeDtypeStruct((B, D), jnp.float32),
    scratch_shapes=[pltpu.VMEM((BLK_I, 1), jnp.float32)],
    compiler_params=pltpu.CompilerParams(
        dimension_semantics=("arbitrary", "arbitrary")),
)


def _scatter_body(newrows_hbm, idx2d_hbm, protos_in, protos_out,
                  idx_v, rows_v, sem):
    del protos_in
    wid = _wid()
    n_chunks = B_PER_W // CHUNK
    pltpu.sync_copy(idx2d_hbm.at[pl.ds(wid * n_chunks, n_chunks), :], idx_v)
    pltpu.sync_copy(newrows_hbm.at[pl.ds(wid * B_PER_W, B_PER_W), :], rows_v)
    descs = []
    for j in range(n_chunks):
        descs.append(pltpu.async_copy(
            rows_v.at[pl.ds(j * CHUNK, CHUNK), :],
            protos_out.at[idx_v.at[j]], sem))
    for d in descs:
        d.wait()


_k_scatter = pl_mpmd._mpmd_map(
    [(plsc.VectorSubcoreMesh(**_MESH), _scatter_body)],
    out_types=jax.ShapeDtypeStruct((G, D), jnp.float32),
    input_output_aliases={2: 0},
    compiler_params=pltpu.CompilerParams(use_tc_tiling_on_sc=False),
    scratch_types=[
        pltpu.VMEM((B_PER_W // CHUNK, CHUNK), jnp.int32),
        pltpu.VMEM((B_PER_W, D), jnp.float32),
        pltpu.SemaphoreType.DMA,
    ],
)


def kernel(features, class_ids, stage_ids, prototypes, counts):
    flat_id = (class_ids.astype(jnp.int32) * S + stage_ids.astype(jnp.int32))
    idx2d = flat_id.reshape(B // CHUNK, CHUNK)
    ids_f = flat_id.astype(jnp.float32)
    protos_flat = prototypes.reshape(G, D)
    counts_flat = counts.reshape(G)
    ones128 = jnp.ones((CHUNK,), jnp.float32)

    protos_copy, counts_new = _k_copy(protos_flat, counts_flat, idx2d, ones128)
    prows = _k_gather(protos_flat, idx2d)
    newrows = _k_mm(ids_f.reshape(B, 1), ids_f.reshape(1, B), features, prows)
    protos_final = _k_scatter(newrows, idx2d, protos_copy)

    return (protos_final.reshape(C, S, D), counts_new.reshape(C, S))


# speedup vs baseline: 4.0432x; 4.0432x over previous
import jax
import jax.numpy as jnp
from jax import lax
from jax.experimental import pallas as pl
from jax.experimental.pallas import tpu as pltpu
from jax.experimental.pallas import tpu_sc as plsc
from jax._src.pallas import mpmd as pl_mpmd

C = 100000
S = 4
D = 64
B = 16384
G = C * S
MOMENTUM = 0.99

NC = 2
NS = 16
NW = NC * NS
CHUNK = 128

B_PER_W = B // NW
N_CHUNKS = B_PER_W // CHUNK

_MESH = dict(core_axis_name="c", subcore_axis_name="s")
_SC_PARAMS = pltpu.CompilerParams(use_tc_tiling_on_sc=False)


def _wid():
    return lax.axis_index("s") * NC + lax.axis_index("c")


def _gather_body(protos_hbm, counts_hbm, idx2d_hbm, rows_out, cnts_out,
                 idx_v, rows_v, cnts_v, sem):
    wid = _wid()
    pltpu.sync_copy(idx2d_hbm.at[pl.ds(wid * N_CHUNKS, N_CHUNKS), :], idx_v)
    descs = []
    for j in range(N_CHUNKS):
        descs.append(pltpu.async_copy(
            protos_hbm.at[idx_v.at[j]],
            rows_v.at[pl.ds(j * CHUNK, CHUNK), :], sem))
        descs.append(pltpu.async_copy(
            counts_hbm.at[idx_v.at[j]], cnts_v.at[j], sem))
    for d in descs:
        d.wait()
    pltpu.sync_copy(rows_v, rows_out.at[pl.ds(wid * B_PER_W, B_PER_W), :])
    pltpu.sync_copy(cnts_v, cnts_out.at[pl.ds(wid * N_CHUNKS, N_CHUNKS), :])


_k_gather = pl.kernel(
    _gather_body,
    out_type=(
        jax.ShapeDtypeStruct((B, D), jnp.float32),
        jax.ShapeDtypeStruct((B // CHUNK, CHUNK), jnp.float32),
    ),
    mesh=plsc.VectorSubcoreMesh(**_MESH),
    compiler_params=_SC_PARAMS,
    scratch_types=[
        pltpu.VMEM((N_CHUNKS, CHUNK), jnp.int32),
        pltpu.VMEM((B_PER_W, D), jnp.float32),
        pltpu.VMEM((N_CHUNKS, CHUNK), jnp.float32),
        pltpu.SemaphoreType.DMA,
    ],
)


BLK_I = 1024
BLK_J = 1024
NI = B // BLK_I
NJ = B // BLK_J


def _mm_body(ids_col, ids_row, feats, prows, pcnts, out, outcnt, cnt):
    j = pl.program_id(1)

    @pl.when(j == 0)
    def _init():
        out[...] = jnp.zeros_like(out)
        cnt[...] = jnp.zeros_like(cnt)

    mask = (ids_col[...] == ids_row[...]).astype(jnp.float32)
    out[...] += jnp.dot(mask, feats[...], preferred_element_type=jnp.float32)
    cnt[...] += jnp.sum(mask, axis=1, keepdims=True)

    @pl.when(j == NJ - 1)
    def _finalize():
        mean = out[...] / cnt[...]
        out[...] = MOMENTUM * prows[...] + (1.0 - MOMENTUM) * mean
        outcnt[...] = pcnts[...] + cnt[...]


_k_mm = pl.pallas_call(
    _mm_body,
    grid=(NI, NJ),
    in_specs=[
        pl.BlockSpec((BLK_I, 1), lambda i, j: (i, 0)),
        pl.BlockSpec((1, BLK_J), lambda i, j: (0, j)),
        pl.BlockSpec((BLK_J, D), lambda i, j: (j, 0)),
        pl.BlockSpec((BLK_I, D), lambda i, j: (i, 0)),
        pl.BlockSpec((BLK_I, 1), lambda i, j: (i, 0)),
    ],
    out_specs=[
        pl.BlockSpec((BLK_I, D), lambda i, j: (i, 0)),
        pl.BlockSpec((BLK_I, 1), lambda i, j: (i, 0)),
    ],
    out_shape=[
        jax.ShapeDtypeStruct((B, D), jnp.float32),
        jax.ShapeDtypeStruct((B, 1), jnp.float32),
    ],
    scratch_shapes=[pltpu.VMEM((BLK_I, 1), jnp.float32)],
    compiler_params=pltpu.CompilerParams(
        dimension_semantics=("arbitrary", "arbitrary")),
)


def _scatter_body(newrows_hbm, newcnts_hbm, idx2d_hbm, protos_io, counts_io,
                  protos_out, counts_out, idx_v, rows_v, cnts_v, sem):
    del protos_io, counts_io
    wid = _wid()
    pltpu.sync_copy(idx2d_hbm.at[pl.ds(wid * N_CHUNKS, N_CHUNKS), :], idx_v)
    pltpu.sync_copy(newrows_hbm.at[pl.ds(wid * B_PER_W, B_PER_W), :], rows_v)
    pltpu.sync_copy(newcnts_hbm.at[pl.ds(wid * N_CHUNKS, N_CHUNKS), :], cnts_v)
    descs = []
    for j in range(N_CHUNKS):
        descs.append(pltpu.async_copy(
            rows_v.at[pl.ds(j * CHUNK, CHUNK), :],
            protos_out.at[idx_v.at[j]], sem))
        descs.append(pltpu.async_copy(
            cnts_v.at[j], counts_out.at[idx_v.at[j]], sem))
    for d in descs:
        d.wait()


_k_scatter = pl_mpmd._mpmd_map(
    [(plsc.VectorSubcoreMesh(**_MESH), _scatter_body)],
    out_types=(
        jax.ShapeDtypeStruct((G, D), jnp.float32),
        jax.ShapeDtypeStruct((G,), jnp.float32),
    ),
    input_output_aliases={3: 0, 4: 1},
    compiler_params=_SC_PARAMS,
    scratch_types=[
        pltpu.VMEM((N_CHUNKS, CHUNK), jnp.int32),
        pltpu.VMEM((B_PER_W, D), jnp.float32),
        pltpu.VMEM((N_CHUNKS, CHUNK), jnp.float32),
        pltpu.SemaphoreType.DMA,
    ],
)


def kernel(features, class_ids, stage_ids, prototypes, counts):
    flat_id = (class_ids.astype(jnp.int32) * S + stage_ids.astype(jnp.int32))
    idx2d = flat_id.reshape(B // CHUNK, CHUNK)
    ids_f = flat_id.astype(jnp.float32)
    protos_flat = prototypes.reshape(G, D)
    counts_flat = counts.reshape(G)

    prows, pcnts = _k_gather(protos_flat, counts_flat, idx2d)
    newrows, newcnts = _k_mm(ids_f.reshape(B, 1), ids_f.reshape(1, B),
                             features, prows, pcnts.reshape(B, 1))
    protos_final, counts_final = _k_scatter(
        newrows, newcnts.reshape(B // CHUNK, CHUNK), idx2d,
        protos_flat, counts_flat)

    return (protos_final.reshape(C, S, D), counts_final.reshape(C, S))
